# baseline (device time: 79313 ns/iter reference)
import jax
import jax.numpy as jnp
from jax import lax
from jax.experimental import pallas as pl
from jax.experimental.pallas import tpu as pltpu

B = 16
NB = 128
BS = 16
H = 16
D = 64
PAGES_LOCAL = 128
KEYS = PAGES_LOCAL * BS
SCALE = D ** -0.5

O_OFF = 0
M_OFF = H * D
L_OFF = H * D + H
COMM_LANES = 1088


def kernel(Q, K, V, bt, lens):
    Q3 = Q.reshape(B, H, D)
    K3 = K.reshape(KEYS, H, D)
    V3 = V.reshape(KEYS, H, D)
    lens2 = lens.reshape(B, 1)

    def body(q_ref, k_ref, v_ref, bt_ref, lens_ref, out_ref,
             comm_ref, send_sem, recv_sem):
        my_x = lax.axis_index("x")
        my_y = lax.axis_index("y")
        nbr = (my_x, 1 - my_y)

        gid = (lax.broadcasted_iota(jnp.int32, (B, PAGES_LOCAL), 1)
               + my_y * PAGES_LOCAL)
        lens_v = lens_ref[:, :]
        cnt = jnp.zeros((B, PAGES_LOCAL), jnp.float32)
        for j in range(NB):
            hit = (bt_ref[:, j:j + 1] == gid) & (j < lens_v)
            cnt = cnt + hit.astype(jnp.float32)

        krow = lax.broadcasted_iota(jnp.int32, (PAGES_LOCAL, KEYS), 0)
        kcol = lax.broadcasted_iota(jnp.int32, (PAGES_LOCAL, KEYS), 1)
        expand = (kcol // BS == krow).astype(jnp.float32)
        c_key = lax.dot_general(cnt, expand, (((1,), (0,)), ((), ())),
                                preferred_element_type=jnp.float32)

        for h in range(H):
            q_h = q_ref[:, h, :].astype(jnp.bfloat16)
            k_h = k_ref[:, h, :].astype(jnp.bfloat16)
            v_h = v_ref[:, h, :].astype(jnp.bfloat16)
            s = lax.dot_general(q_h, k_h, (((1,), (1,)), ((), ())),
                                preferred_element_type=jnp.float32) * SCALE
            m_h = jnp.max(s, axis=1, keepdims=True)
            w = jnp.exp(s - m_h) * c_key
            l_h = jnp.sum(w, axis=1, keepdims=True)
            o_h = lax.dot_general(w.astype(jnp.bfloat16), v_h,
                                  (((1,), (0,)), ((), ())),
                                  preferred_element_type=jnp.float32)
            comm_ref[0, :, O_OFF + h * D:O_OFF + (h + 1) * D] = o_h
            comm_ref[0, :, M_OFF + h:M_OFF + h + 1] = m_h
            comm_ref[0, :, L_OFF + h:L_OFF + h + 1] = l_h

        barrier = pltpu.get_barrier_semaphore()
        pl.semaphore_signal(barrier, 1, device_id=nbr,
                            device_id_type=pl.DeviceIdType.MESH)
        pl.semaphore_wait(barrier, 1)

        rdma = pltpu.make_async_remote_copy(
            src_ref=comm_ref.at[0],
            dst_ref=comm_ref.at[1],
            send_sem=send_sem,
            recv_sem=recv_sem,
            device_id=nbr,
            device_id_type=pl.DeviceIdType.MESH,
        )
        rdma.start()
        rdma.wait()

        m_a = comm_ref[0, :, M_OFF:M_OFF + H]
        l_a = comm_ref[0, :, L_OFF:L_OFF + H]
        m_b = comm_ref[1, :, M_OFF:M_OFF + H]
        l_b = comm_ref[1, :, L_OFF:L_OFF + H]
        m_n = jnp.maximum(m_a, m_b)
        alpha = jnp.exp(m_a - m_n)
        beta = jnp.exp(m_b - m_n)
        inv_l = 1.0 / (l_a * alpha + l_b * beta)
        for h in range(H):
            o_a = comm_ref[0, :, O_OFF + h * D:O_OFF + (h + 1) * D]
            o_b = comm_ref[1, :, O_OFF + h * D:O_OFF + (h + 1) * D]
            o = (o_a * alpha[:, h:h + 1] + o_b * beta[:, h:h + 1])
            out_ref[:, h, :] = o * inv_l[:, h:h + 1]

    out = pl.pallas_call(
        body,
        out_shape=jax.ShapeDtypeStruct((B, H, D), jnp.float32),
        in_specs=[pl.BlockSpec(memory_space=pltpu.VMEM)] * 5,
        out_specs=pl.BlockSpec(memory_space=pltpu.VMEM),
        scratch_shapes=[
            pltpu.VMEM((2, B, COMM_LANES), jnp.float32),
            pltpu.SemaphoreType.DMA,
            pltpu.SemaphoreType.DMA,
        ],
        compiler_params=pltpu.CompilerParams(collective_id=0),
    )(Q3, K3, V3, bt, lens2)
    return out.reshape(B, 1, H, D)
